# speedup vs baseline: 3.7023x; 3.7023x over previous
import jax
import jax.numpy as jnp
from jax.experimental import pallas as pl
from jax.experimental.pallas import tpu as pltpu
from jax.experimental.pallas import tpu_sc as plsc

DIM = 128
WINDOW = 256


def kernel(tgt, emb_weight):
    batch, hist = tgt.shape
    n = batch * hist
    idx = tgt.reshape(1, n).astype(jnp.int32)

    mesh = plsc.VectorSubcoreMesh(core_axis_name="core",
                                  subcore_axis_name="subcore")

    @pl.kernel(out_type=jax.ShapeDtypeStruct((n, DIM), emb_weight.dtype),
               mesh=mesh)
    def gather_kernel(table_hbm, idx_hbm, out_hbm):
        def body(idx_vmem, out_vmem):
            pltpu.sync_copy(table_hbm.at[idx_vmem.at[0]], out_vmem)

        pltpu.emit_pipeline(
            body,
            grid=(n // WINDOW,),
            in_specs=[pl.BlockSpec((1, WINDOW), index_map=lambda i: (0, i))],
            out_specs=[pl.BlockSpec((WINDOW, DIM), index_map=lambda i: (i, 0))],
            core_axis_name=("core", "subcore"),
            dimension_semantics=(pltpu.PARALLEL,),
        )(idx_hbm, out_hbm)

    out = gather_kernel(emb_weight, idx)
    return out.reshape(batch, hist, DIM)


# speedup vs baseline: 3.7062x; 1.0011x over previous
import jax
import jax.numpy as jnp
from jax.experimental import pallas as pl
from jax.experimental.pallas import tpu as pltpu
from jax.experimental.pallas import tpu_sc as plsc

DIM = 128
WINDOW = 256


def kernel(tgt, emb_weight):
    batch, hist = tgt.shape
    n = batch * hist
    idx = tgt.reshape(1, n).astype(jnp.int32)

    mesh = plsc.VectorSubcoreMesh(core_axis_name="core",
                                  subcore_axis_name="subcore")

    nsplit = 4
    sub = WINDOW // nsplit

    @pl.kernel(out_type=jax.ShapeDtypeStruct((n, DIM), emb_weight.dtype),
               mesh=mesh,
               scratch_types=[pltpu.SemaphoreType.DMA])
    def gather_kernel(table_hbm, idx_hbm, out_hbm, sem):
        def body(idx_vmem, out_vmem):
            copies = [
                pltpu.async_copy(
                    table_hbm.at[idx_vmem.at[0, pl.ds(j * sub, sub)]],
                    out_vmem.at[pl.ds(j * sub, sub)],
                    sem,
                )
                for j in range(nsplit)
            ]
            for c in copies:
                c.wait()

        pltpu.emit_pipeline(
            body,
            grid=(n // WINDOW,),
            in_specs=[pl.BlockSpec((1, WINDOW), index_map=lambda i: (0, i))],
            out_specs=[pl.BlockSpec((WINDOW, DIM), index_map=lambda i: (i, 0))],
            core_axis_name=("core", "subcore"),
            dimension_semantics=(pltpu.PARALLEL,),
        )(idx_hbm, out_hbm)

    out = gather_kernel(emb_weight, idx)
    return out.reshape(batch, hist, DIM)
